# Initial kernel scaffold; baseline (speedup 1.0000x reference)
import jax
import jax.numpy as jnp
from jax.experimental import pallas as pl


def kernel(item_id, feat_brand, feat_cate, count, noise, item_emb_table, brand_table, cate_table, codebook, me_w1, me_b1, me_w2, me_b2, lv_w1, lv_b1, lv_w2, lv_b2, mp_w1, mp_b1, mp_w2, mp_b2, lp_w1, lp_b1, lp_w2, lp_b2, dec_w1, dec_b1, dec_w2, dec_b2, head_w, head_b):
    raise NotImplementedError("write your pallas kernel here")


# speedup vs baseline: 1.0856x; 1.0856x over previous
import functools

import jax
import jax.numpy as jnp
from jax import lax
from jax.experimental import pallas as pl
from jax.experimental.pallas import tpu as pltpu
from jax.experimental.pallas import tpu_sc as plsc

B = 16384
EMB = 16
CB_SIZE = 32
CB_DIM = 8

_NC = 2
_NS = 16
_NW = _NC * _NS
_BPW = B // _NW


def _sc_gather_body(item_t, brand_t, cate_t, idx_i, idx_b, idx_c,
                    out_i, out_b, out_c,
                    iv0, iv1, iv2, rv0, rv1, rv2, s0, s1, s2):
    wid = lax.axis_index("s") * _NC + lax.axis_index("c")
    base = wid * _BPW
    pltpu.sync_copy(idx_i.at[pl.ds(base, _BPW)], iv0)
    pltpu.sync_copy(idx_b.at[pl.ds(base, _BPW)], iv1)
    pltpu.sync_copy(idx_c.at[pl.ds(base, _BPW)], iv2)
    c0 = pltpu.async_copy(item_t.at[iv0], rv0, s0)
    c1 = pltpu.async_copy(brand_t.at[iv1], rv1, s1)
    c2 = pltpu.async_copy(cate_t.at[iv2], rv2, s2)
    c0.wait()
    c1.wait()
    c2.wait()
    pltpu.sync_copy(rv0, out_i.at[pl.ds(base, _BPW)])
    pltpu.sync_copy(rv1, out_b.at[pl.ds(base, _BPW)])
    pltpu.sync_copy(rv2, out_c.at[pl.ds(base, _BPW)])


@jax.jit
def _sc_gather(item_t, brand_t, cate_t, idx_i, idx_b, idx_c):
    mesh = plsc.VectorSubcoreMesh(core_axis_name="c", subcore_axis_name="s")
    row = jax.ShapeDtypeStruct((B, EMB), jnp.float32)
    run = pl.kernel(
        _sc_gather_body,
        mesh=mesh,
        compiler_params=pltpu.CompilerParams(use_tc_tiling_on_sc=False),
        out_type=(row, row, row),
        scratch_types=[
            pltpu.VMEM((_BPW,), jnp.int32),
            pltpu.VMEM((_BPW,), jnp.int32),
            pltpu.VMEM((_BPW,), jnp.int32),
            pltpu.VMEM((_BPW, EMB), jnp.float32),
            pltpu.VMEM((_BPW, EMB), jnp.float32),
            pltpu.VMEM((_BPW, EMB), jnp.float32),
            pltpu.SemaphoreType.DMA,
            pltpu.SemaphoreType.DMA,
            pltpu.SemaphoreType.DMA,
        ],
    )
    return run(item_t, brand_t, cate_t, idx_i, idx_b, idx_c)


_BLK = 2048
_NBLK = B // _BLK


def _dense_body(item_emb_ref, brand_ref, cate_ref, count_ref, noise_ref,
                cb_ref, cbt_ref,
                me_w1, me_b1, me_w2, me_b2,
                lv_w1, lv_b1, lv_w2, lv_b2,
                mp_w1a, mp_w1b, mp_b1, mp_w2, mp_b2,
                lp_w1a, lp_w1b, lp_b1, lp_w2, lp_b2,
                dec_w1a, dec_w1b, dec_b1, dec_w2, dec_b2,
                head_w, head_b,
                recon_ref, reg_ref, target_ref, vq_ref, perp_ref,
                counts_ref):
    f32 = jnp.float32
    step = pl.program_id(0)
    item = item_emb_ref[...]
    brand = brand_ref[...]
    cate = cate_ref[...]
    noise = noise_ref[...]
    count = count_ref[...]

    def mlp2(x, w1, b1, w2, b2):
        h = jnp.maximum(jnp.dot(x, w1[...], preferred_element_type=f32)
                        + b1[...], 0.0)
        return jnp.dot(h, w2[...], preferred_element_type=f32) + b2[...]

    mean = mlp2(item, me_w1, me_b1, me_w2, me_b2)
    log_v = mlp2(item, lv_w1, lv_b1, lv_w2, lv_b2)

    def mlp2_side(w1a, w1b, b1, w2, b2):
        h = (jnp.dot(brand, w1a[...], preferred_element_type=f32)
             + jnp.dot(cate, w1b[...], preferred_element_type=f32) + b1[...])
        h = jnp.maximum(h, 0.0)
        return jnp.dot(h, w2[...], preferred_element_type=f32) + b2[...]

    mean_p = mlp2_side(mp_w1a, mp_w1b, mp_b1, mp_w2, mp_b2)
    log_v_p = mlp2_side(lp_w1a, lp_w1b, lp_b1, lp_w2, lp_b2)

    p1 = jnp.sum(jnp.square(mean - mean_p), axis=1)
    p2 = jnp.sum(jnp.square(jnp.exp(log_v * 0.5) - jnp.exp(log_v_p * 0.5)),
                 axis=1)
    reg_part = jnp.reshape(jnp.sum(p1 + p2), (1, 1))

    z = mean + jnp.exp(log_v * 0.5) * noise

    cbt = cbt_ref[...]
    cb2 = jnp.sum(cbt * cbt, axis=0, keepdims=True)
    score = cb2 - 2.0 * jnp.dot(z, cbt, preferred_element_type=f32)
    m = jnp.min(score, axis=1, keepdims=True)
    iota = lax.broadcasted_iota(jnp.int32, score.shape, 1)
    big = jnp.int32(CB_SIZE)
    idx = jnp.min(jnp.where(score == m, iota, big), axis=1, keepdims=True)
    one_hot = (iota == idx).astype(f32)
    z_q = jnp.dot(one_hot, cb_ref[...], preferred_element_type=f32)

    vq_part = jnp.reshape(jnp.sum(jnp.square(z_q - z)), (1, 1))
    counts_part = jnp.reshape(jnp.sum(one_hot, axis=0), (1, CB_SIZE))

    h = (jnp.dot(z, dec_w1a[...], preferred_element_type=f32)
         + count * dec_w1b[...] + dec_b1[...])
    h = jnp.maximum(h, 0.0)
    pred = jnp.dot(h, dec_w2[...], preferred_element_type=f32) + dec_b2[...]

    recon_part = jnp.reshape(jnp.sum(jnp.square(pred - item)), (1, 1))

    logit = jnp.dot(pred, head_w[...], preferred_element_type=f32) + head_b[...]
    target_ref[...] = 1.0 / (1.0 + jnp.exp(-logit))

    @pl.when(step == 0)
    def _init():
        reg_ref[...] = reg_part
        vq_ref[...] = vq_part
        recon_ref[...] = recon_part
        counts_ref[...] = counts_part

    @pl.when(step > 0)
    def _acc():
        reg_ref[...] += reg_part
        vq_ref[...] += vq_part
        recon_ref[...] += recon_part
        counts_ref[...] += counts_part

    @pl.when(step == _NBLK - 1)
    def _finalize():
        vq_ref[...] = vq_ref[...] * (1.25 / (B * CB_DIM))
        recon_ref[...] = recon_ref[...] * (1.0 / B)
        probs = counts_ref[...] * (1.0 / B)
        ent = jnp.sum(probs * jnp.log(probs + 1e-10))
        perp_ref[...] = jnp.reshape(jnp.exp(-ent), (1, 1))


@jax.jit
def _dense(item_emb, brand_emb, cate_emb, count, noise, codebook,
           me_w1, me_b1, me_w2, me_b2,
           lv_w1, lv_b1, lv_w2, lv_b2,
           mp_w1, mp_b1, mp_w2, mp_b2,
           lp_w1, lp_b1, lp_w2, lp_b2,
           dec_w1, dec_b1, dec_w2, dec_b2,
           head_w, head_b):
    r2 = lambda b: b.reshape(1, -1)
    scalar = jax.ShapeDtypeStruct((1, 1), jnp.float32)
    out_shape = (scalar, scalar,
                 jax.ShapeDtypeStruct((B, 1), jnp.float32),
                 scalar, scalar)
    blk = lambda i: (i, 0)
    cst = lambda i: (0, 0)
    wspec = pl.BlockSpec(index_map=cst)
    row_spec = lambda w: pl.BlockSpec((_BLK, w), blk)
    scal_spec = pl.BlockSpec((1, 1), cst)
    outs = pl.pallas_call(
        _dense_body,
        grid=(_NBLK,),
        in_specs=[row_spec(EMB), row_spec(EMB), row_spec(EMB),
                  row_spec(1), row_spec(CB_DIM)] + [wspec] * 27,
        out_specs=(scal_spec, scal_spec, pl.BlockSpec((_BLK, 1), blk),
                   scal_spec, scal_spec),
        scratch_shapes=[pltpu.VMEM((1, CB_SIZE), jnp.float32)],
        out_shape=out_shape,
    )(item_emb, brand_emb, cate_emb, count, noise,
      codebook, codebook.T,
      me_w1, r2(me_b1), me_w2, r2(me_b2),
      lv_w1, r2(lv_b1), lv_w2, r2(lv_b2),
      mp_w1[:EMB], mp_w1[EMB:], r2(mp_b1), mp_w2, r2(mp_b2),
      lp_w1[:EMB], lp_w1[EMB:], r2(lp_b1), lp_w2, r2(lp_b2),
      dec_w1[:CB_DIM], dec_w1[CB_DIM:], r2(dec_b1), dec_w2, r2(dec_b2),
      head_w, r2(head_b))
    recon, reg, target, vq, perp = outs
    return (recon[0, 0], reg[0, 0], target, vq[0, 0], perp[0, 0])


def kernel(item_id, feat_brand, feat_cate, count, noise, item_emb_table,
           brand_table, cate_table, codebook,
           me_w1, me_b1, me_w2, me_b2,
           lv_w1, lv_b1, lv_w2, lv_b2,
           mp_w1, mp_b1, mp_w2, mp_b2,
           lp_w1, lp_b1, lp_w2, lp_b2,
           dec_w1, dec_b1, dec_w2, dec_b2,
           head_w, head_b):
    item_emb, brand_emb, cate_emb = _sc_gather(
        item_emb_table, brand_table, cate_table,
        item_id.astype(jnp.int32), feat_brand.astype(jnp.int32),
        feat_cate.astype(jnp.int32))
    return _dense(item_emb, brand_emb, cate_emb, count, noise, codebook,
                  me_w1, me_b1, me_w2, me_b2,
                  lv_w1, lv_b1, lv_w2, lv_b2,
                  mp_w1, mp_b1, mp_w2, mp_b2,
                  lp_w1, lp_b1, lp_w2, lp_b2,
                  dec_w1, dec_b1, dec_w2, dec_b2,
                  head_w, head_b)
